# Initial kernel scaffold; baseline (speedup 1.0000x reference)
import jax
import jax.numpy as jnp
from jax.experimental import pallas as pl


def kernel(x, edge_index, edge_attr, a, e1_w1, e1_b1, e1_w2, e1_b2, root1, bias1, e2_w1, e2_b1, e2_w2, e2_b2, root2, bias2, proj_w, proj_b, mw1, mb1, mw2, mb2, mw3, mb3):
    raise NotImplementedError("write your pallas kernel here")


# speedup vs baseline: 1.6193x; 1.6193x over previous
import functools

import jax
import jax.numpy as jnp
from jax import lax
from jax.experimental import pallas as pl
from jax.experimental.pallas import tpu as pltpu
from jax.experimental.pallas import tpu_sc as plsc

_N = 10000
_E = 50000
_NODE_IN = 32
_EDGE_IN = 16
_HID = 64
_G_DIM = 64
_N_ACT = 16
_B = 1024
_MLP_HID = 128

_NW = 32
_CHUNK = 112
_NCHUNK = 14
_EPW = _CHUNK * _NCHUNK
_E_PAD = _NW * _EPW
_N_PAD = 10240
_RPT = _N_PAD // 16
_ET = 512
_W1C = _HID + 16


def _sc_mesh():
    return plsc.VectorSubcoreMesh(core_axis_name="c", subcore_axis_name="s")


def _make_gather(d):
    @functools.partial(
        pl.kernel,
        out_type=jax.ShapeDtypeStruct((_E_PAD, d), jnp.float32),
        mesh=_sc_mesh(),
        scratch_types=[
            pltpu.VMEM((_NCHUNK, _CHUNK), jnp.int32),
            pltpu.VMEM((_EPW, d), jnp.float32),
            pltpu.SemaphoreType.DMA,
        ],
        compiler_params=pltpu.CompilerParams(use_tc_tiling_on_sc=False),
    )
    def gather_k(tbl_hbm, idx_hbm, out_hbm, idx_v, rows_v, sem):
        wid = lax.axis_index("s") * 2 + lax.axis_index("c")
        pltpu.sync_copy(idx_hbm.at[wid], idx_v)
        for j in range(_NCHUNK):
            pltpu.async_copy(
                tbl_hbm.at[idx_v.at[j]],
                rows_v.at[pl.ds(j * _CHUNK, _CHUNK)],
                sem,
            ).wait()
        pltpu.sync_copy(rows_v, out_hbm.at[pl.ds(wid * _EPW, _EPW)])

    return gather_k


def _make_scatter(w):
    @functools.partial(
        pl.kernel,
        out_type=jax.ShapeDtypeStruct((2, _N_PAD, w), jnp.float32),
        mesh=_sc_mesh(),
        scratch_types=[
            pltpu.VMEM((_NCHUNK, _CHUNK), jnp.int32),
            pltpu.VMEM((_CHUNK, w), jnp.float32),
            pltpu.VMEM_SHARED((_N_PAD, w), jnp.float32),
            pltpu.SemaphoreType.DMA,
        ],
        compiler_params=pltpu.CompilerParams(use_tc_tiling_on_sc=False),
    )
    def scatter_k(msg_hbm, dst_hbm, zeros_hbm, out_hbm, idx_v, buf_v, acc_sh, sem):
        cid = lax.axis_index("c")
        sid = lax.axis_index("s")
        wid = sid * 2 + cid
        pltpu.sync_copy(zeros_hbm, acc_sh.at[pl.ds(sid * _RPT, _RPT)])
        plsc.subcore_barrier()
        pltpu.sync_copy(dst_hbm.at[wid], idx_v)
        for j in range(_NCHUNK):
            pltpu.sync_copy(
                msg_hbm.at[pl.ds(wid * _EPW + j * _CHUNK, _CHUNK)], buf_v
            )
            pltpu.sync_copy(buf_v, acc_sh.at[idx_v.at[j]], add=True)
        plsc.subcore_barrier()
        pltpu.sync_copy(
            acc_sh.at[pl.ds(sid * _RPT, _RPT)],
            out_hbm.at[cid, pl.ds(sid * _RPT, _RPT)],
        )

    return scatter_k


def _msg_body(in_ch, with_flag, ea_ref, xs_ref, w1_ref, b1_ref, w2ik_ref,
              b2r_ref, out_ref):
    h = jnp.maximum(
        jnp.dot(ea_ref[...], w1_ref[...], preferred_element_type=jnp.float32)
        + b1_ref[...], 0.0)
    xs = xs_ref[...]
    u = (xs[:, :, None] * h[:, None, :]).reshape(_ET, in_ch * _HID)
    msg = jnp.dot(u, w2ik_ref[...], preferred_element_type=jnp.float32)
    msg = msg + jnp.dot(xs, b2r_ref[...], preferred_element_type=jnp.float32)
    out_ref[:, :_HID] = msg
    if with_flag:
        i = pl.program_id(0)
        rowid = i * _ET + lax.broadcasted_iota(jnp.int32, (_ET, 16), 0)
        out_ref[:, _HID:] = jnp.where(rowid < _E, 1.0, 0.0)


def _edge_messages(ea, xs, w1, b1, w2ik, b2r, in_ch, with_flag):
    outw = _W1C if with_flag else _HID
    grid = (_E_PAD // _ET,)
    return pl.pallas_call(
        functools.partial(_msg_body, in_ch, with_flag),
        grid=grid,
        in_specs=[
            pl.BlockSpec((_ET, _EDGE_IN), lambda i: (i, 0)),
            pl.BlockSpec((_ET, in_ch), lambda i: (i, 0)),
            pl.BlockSpec((_EDGE_IN, _HID), lambda i: (0, 0)),
            pl.BlockSpec((1, _HID), lambda i: (0, 0)),
            pl.BlockSpec((in_ch * _HID, _HID), lambda i: (0, 0)),
            pl.BlockSpec((in_ch, _HID), lambda i: (0, 0)),
        ],
        out_specs=pl.BlockSpec((_ET, outw), lambda i: (i, 0)),
        out_shape=jax.ShapeDtypeStruct((_E_PAD, outw), jnp.float32),
    )(ea, xs, w1, b1, w2ik, b2r)


def _node_update1_body(s_ref, x_ref, r_ref, b_ref, h_ref, inv_ref):
    s = s_ref[0] + s_ref[1]
    cnt = s[:, _HID:_HID + 1]
    inv = 1.0 / jnp.maximum(cnt, 1.0)
    mean = s[:, :_HID] * inv
    h = jnp.maximum(
        mean + jnp.dot(x_ref[...], r_ref[...], preferred_element_type=jnp.float32)
        + b_ref[...], 0.0)
    h_ref[...] = h
    inv_ref[...] = jnp.broadcast_to(inv, (_N_PAD, 8))


def _head_body(s_ref, h1_ref, inv_ref, r_ref, b_ref, pw_ref, pb_ref, a_ref,
               mw1_ref, mb1_ref, mw2_ref, mb2_ref, mw3_ref, mb3_ref, out_ref):
    s = s_ref[0] + s_ref[1]
    mean2 = s * inv_ref[:, 0:1]
    h2 = jnp.maximum(
        mean2
        + jnp.dot(h1_ref[...], r_ref[...], preferred_element_type=jnp.float32)
        + b_ref[...], 0.0)
    rid = lax.broadcasted_iota(jnp.int32, (_N_PAD, _HID), 0)
    h2 = jnp.where(rid < _N, h2, 0.0)
    pooled = jnp.sum(h2, axis=0, keepdims=True) * (1.0 / _N)
    g = jnp.dot(pooled, pw_ref[...], preferred_element_type=jnp.float32) + pb_ref[...]
    z = (jnp.dot(a_ref[...], mw1_ref[:_N_ACT, :], preferred_element_type=jnp.float32)
         + jnp.dot(g, mw1_ref[_N_ACT:, :], preferred_element_type=jnp.float32)
         + mb1_ref[...])
    z = jnp.maximum(z, 0.0)
    z = jnp.maximum(
        jnp.dot(z, mw2_ref[...], preferred_element_type=jnp.float32) + mb2_ref[...],
        0.0)
    out_ref[...] = (
        jnp.dot(z, mw3_ref[...], preferred_element_type=jnp.float32) + mb3_ref[...])


def kernel(x, edge_index, edge_attr, a,
           e1_w1, e1_b1, e1_w2, e1_b2, root1, bias1,
           e2_w1, e2_b1, e2_w2, e2_b2, root2, bias2,
           proj_w, proj_b, mw1, mb1, mw2, mb2, mw3, mb3):
    f32 = jnp.float32
    pad = _E_PAD - _E
    src = jnp.concatenate([edge_index[0], jnp.zeros((pad,), jnp.int32)])
    src = src.reshape(_NW, _NCHUNK, _CHUNK)
    dst = jnp.concatenate([edge_index[1], jnp.full((pad,), _N, jnp.int32)])
    dst = dst.reshape(_NW, _NCHUNK, _CHUNK)
    ea = jnp.concatenate([edge_attr, jnp.zeros((pad, _EDGE_IN), f32)])
    x_pad = jnp.concatenate([x, jnp.zeros((_N_PAD - _N, _NODE_IN), f32)])

    w2ik1 = e1_w2.reshape(_HID, _NODE_IN, _HID).transpose(1, 0, 2)
    w2ik1 = w2ik1.reshape(_NODE_IN * _HID, _HID)
    b2r1 = e1_b2.reshape(_NODE_IN, _HID)
    w2ik2 = e2_w2.reshape(_HID, _HID, _HID).transpose(1, 0, 2)
    w2ik2 = w2ik2.reshape(_HID * _HID, _HID)
    b2r2 = e2_b2.reshape(_HID, _HID)

    zeros1 = jnp.zeros((_RPT, _W1C), f32)
    zeros2 = jnp.zeros((_RPT, _HID), f32)

    xs = _make_gather(_NODE_IN)(x, src)
    msg1 = _edge_messages(ea, xs, e1_w1, e1_b1.reshape(1, _HID), w2ik1, b2r1,
                          _NODE_IN, True)
    s1 = _make_scatter(_W1C)(msg1, dst, zeros1)
    h1, inv = pl.pallas_call(
        _node_update1_body,
        in_specs=[
            pl.BlockSpec((2, _N_PAD, _W1C), lambda: (0, 0, 0)),
            pl.BlockSpec((_N_PAD, _NODE_IN), lambda: (0, 0)),
            pl.BlockSpec((_NODE_IN, _HID), lambda: (0, 0)),
            pl.BlockSpec((1, _HID), lambda: (0, 0)),
        ],
        out_specs=[
            pl.BlockSpec((_N_PAD, _HID), lambda: (0, 0)),
            pl.BlockSpec((_N_PAD, 8), lambda: (0, 0)),
        ],
        out_shape=[
            jax.ShapeDtypeStruct((_N_PAD, _HID), f32),
            jax.ShapeDtypeStruct((_N_PAD, 8), f32),
        ],
    )(s1, x_pad, root1, bias1.reshape(1, _HID))

    h1s = _make_gather(_HID)(h1, src)
    msg2 = _edge_messages(ea, h1s, e2_w1, e2_b1.reshape(1, _HID), w2ik2, b2r2,
                          _HID, False)
    s2 = _make_scatter(_HID)(msg2, dst, zeros2)

    full = lambda *shape: pl.BlockSpec(shape, lambda: tuple(0 for _ in shape))
    out = pl.pallas_call(
        _head_body,
        in_specs=[
            full(2, _N_PAD, _HID),
            full(_N_PAD, _HID),
            full(_N_PAD, 8),
            full(_HID, _HID),
            full(1, _HID),
            full(_HID, _G_DIM),
            full(1, _G_DIM),
            full(_B, _N_ACT),
            full(_N_ACT + _G_DIM, _MLP_HID),
            full(1, _MLP_HID),
            full(_MLP_HID, _MLP_HID),
            full(1, _MLP_HID),
            full(_MLP_HID, 1),
            full(1, 1),
        ],
        out_specs=full(_B, 1),
        out_shape=jax.ShapeDtypeStruct((_B, 1), f32),
    )(s2, h1, inv, root2, bias2.reshape(1, _HID), proj_w,
      proj_b.reshape(1, _G_DIM), a, mw1, mb1.reshape(1, _MLP_HID), mw2,
      mb2.reshape(1, _MLP_HID), mw3, mb3.reshape(1, 1))
    return out


# speedup vs baseline: 3.0797x; 1.9019x over previous
import functools

import jax
import jax.numpy as jnp
from jax import lax
from jax.experimental import pallas as pl
from jax.experimental.pallas import tpu as pltpu
from jax.experimental.pallas import tpu_sc as plsc

_N = 10000
_E = 50000
_NODE_IN = 32
_EDGE_IN = 16
_HID = 64
_G_DIM = 64
_N_ACT = 16
_B = 1024
_MLP_HID = 128

_NW = 32
_CHUNK = 112
_NCHUNK = 14
_EPW = _CHUNK * _NCHUNK
_E_PAD = _NW * _EPW
_N_PAD = 10240
_RPT = _N_PAD // 16
_ET = 512
_W1C = _HID + 16


def _sc_mesh():
    return plsc.VectorSubcoreMesh(core_axis_name="c", subcore_axis_name="s")


def _make_gather(d):
    @functools.partial(
        pl.kernel,
        out_type=jax.ShapeDtypeStruct((_E_PAD, d), jnp.float32),
        mesh=_sc_mesh(),
        scratch_types=[
            pltpu.VMEM((_NCHUNK, _CHUNK), jnp.int32),
            pltpu.VMEM((_EPW, d), jnp.float32),
            pltpu.SemaphoreType.DMA,
        ],
        compiler_params=pltpu.CompilerParams(use_tc_tiling_on_sc=False),
    )
    def gather_k(tbl_hbm, idx_hbm, out_hbm, idx_v, rows_v, sem):
        wid = lax.axis_index("s") * 2 + lax.axis_index("c")
        pltpu.sync_copy(idx_hbm.at[wid], idx_v)
        for j in range(_NCHUNK):
            pltpu.async_copy(
                tbl_hbm.at[idx_v.at[j]],
                rows_v.at[pl.ds(j * _CHUNK, _CHUNK)],
                sem,
            ).wait()
        pltpu.sync_copy(rows_v, out_hbm.at[pl.ds(wid * _EPW, _EPW)])

    return gather_k


def _make_scatter(w):
    @functools.partial(
        pl.kernel,
        out_type=jax.ShapeDtypeStruct((2, _N_PAD, w), jnp.float32),
        mesh=_sc_mesh(),
        scratch_types=[
            pltpu.VMEM((_NCHUNK, _CHUNK), jnp.int32),
            pltpu.VMEM((_CHUNK, w), jnp.float32),
            pltpu.VMEM_SHARED((_N_PAD, w), jnp.float32),
            pltpu.SemaphoreType.DMA,
        ],
        compiler_params=pltpu.CompilerParams(use_tc_tiling_on_sc=False),
    )
    def scatter_k(msg_hbm, dst_hbm, zeros_hbm, out_hbm, idx_v, buf_v, acc_sh, sem):
        cid = lax.axis_index("c")
        sid = lax.axis_index("s")
        wid = sid * 2 + cid
        pltpu.sync_copy(zeros_hbm, acc_sh.at[pl.ds(sid * _RPT, _RPT)])
        plsc.subcore_barrier()
        pltpu.sync_copy(dst_hbm.at[wid], idx_v)
        for j in range(_NCHUNK):
            pltpu.sync_copy(
                msg_hbm.at[pl.ds(wid * _EPW + j * _CHUNK, _CHUNK)], buf_v
            )
            pltpu.sync_copy(buf_v, acc_sh.at[idx_v.at[j]], add=True)
        plsc.subcore_barrier()
        pltpu.sync_copy(
            acc_sh.at[pl.ds(sid * _RPT, _RPT)],
            out_hbm.at[cid, pl.ds(sid * _RPT, _RPT)],
        )

    return scatter_k


def _msg_body(in_ch, with_flag, ea_ref, xs_ref, w1_ref, b1_ref, w2_ref,
              b2r_ref, rmat_ref, out_ref):
    h = jnp.maximum(
        jnp.dot(ea_ref[...], w1_ref[...], preferred_element_type=jnp.float32)
        + b1_ref[...], 0.0)
    xs = xs_ref[...]
    hw = jnp.dot(h, w2_ref[...], preferred_element_type=jnp.float32)
    xs2 = jnp.dot(xs, rmat_ref[...], preferred_element_type=jnp.float32)
    u = xs2 * hw
    w = in_ch * _HID
    while w > 2 * _HID:
        w //= 2
        u = u[:, :w] + u[:, w:]
    msg = u[:, :_HID] + u[:, _HID:]
    msg = msg + jnp.dot(xs, b2r_ref[...], preferred_element_type=jnp.float32)
    out_ref[:, :_HID] = msg
    if with_flag:
        i = pl.program_id(0)
        rowid = i * _ET + lax.broadcasted_iota(jnp.int32, (_ET, 16), 0)
        out_ref[:, _HID:] = jnp.where(rowid < _E, 1.0, 0.0)


def _edge_messages(ea, xs, w1, b1, w2ik, b2r, in_ch, with_flag):
    outw = _W1C if with_flag else _HID
    grid = (_E_PAD // _ET,)
    return pl.pallas_call(
        functools.partial(_msg_body, in_ch, with_flag),
        grid=grid,
        in_specs=[
            pl.BlockSpec((_ET, _EDGE_IN), lambda i: (i, 0)),
            pl.BlockSpec((_ET, in_ch), lambda i: (i, 0)),
            pl.BlockSpec((_EDGE_IN, _HID), lambda i: (0, 0)),
            pl.BlockSpec((1, _HID), lambda i: (0, 0)),
            pl.BlockSpec((_HID, in_ch * _HID), lambda i: (0, 0)),
            pl.BlockSpec((in_ch, _HID), lambda i: (0, 0)),
            pl.BlockSpec((in_ch, in_ch * _HID), lambda i: (0, 0)),
        ],
        out_specs=pl.BlockSpec((_ET, outw), lambda i: (i, 0)),
        out_shape=jax.ShapeDtypeStruct((_E_PAD, outw), jnp.float32),
    )(ea, xs, w1, b1, w2ik, b2r, jnp.repeat(jnp.eye(in_ch, dtype=jnp.float32),
                                            _HID, axis=1))


def _node_update1_body(s_ref, x_ref, r_ref, b_ref, h_ref, inv_ref):
    s = s_ref[0] + s_ref[1]
    cnt = s[:, _HID:_HID + 1]
    inv = 1.0 / jnp.maximum(cnt, 1.0)
    mean = s[:, :_HID] * inv
    h = jnp.maximum(
        mean + jnp.dot(x_ref[...], r_ref[...], preferred_element_type=jnp.float32)
        + b_ref[...], 0.0)
    h_ref[...] = h
    inv_ref[...] = jnp.broadcast_to(inv, (_N_PAD, 8))


def _head_body(s_ref, h1_ref, inv_ref, r_ref, b_ref, pw_ref, pb_ref, a_ref,
               mw1_ref, mb1_ref, mw2_ref, mb2_ref, mw3_ref, mb3_ref, out_ref):
    s = s_ref[0] + s_ref[1]
    mean2 = s * inv_ref[:, 0:1]
    h2 = jnp.maximum(
        mean2
        + jnp.dot(h1_ref[...], r_ref[...], preferred_element_type=jnp.float32)
        + b_ref[...], 0.0)
    rid = lax.broadcasted_iota(jnp.int32, (_N_PAD, _HID), 0)
    h2 = jnp.where(rid < _N, h2, 0.0)
    pooled = jnp.sum(h2, axis=0, keepdims=True) * (1.0 / _N)
    g = jnp.dot(pooled, pw_ref[...], preferred_element_type=jnp.float32) + pb_ref[...]
    z = (jnp.dot(a_ref[...], mw1_ref[:_N_ACT, :], preferred_element_type=jnp.float32)
         + jnp.dot(g, mw1_ref[_N_ACT:, :], preferred_element_type=jnp.float32)
         + mb1_ref[...])
    z = jnp.maximum(z, 0.0)
    z = jnp.maximum(
        jnp.dot(z, mw2_ref[...], preferred_element_type=jnp.float32) + mb2_ref[...],
        0.0)
    out_ref[...] = (
        jnp.dot(z, mw3_ref[...], preferred_element_type=jnp.float32) + mb3_ref[...])


def kernel(x, edge_index, edge_attr, a,
           e1_w1, e1_b1, e1_w2, e1_b2, root1, bias1,
           e2_w1, e2_b1, e2_w2, e2_b2, root2, bias2,
           proj_w, proj_b, mw1, mb1, mw2, mb2, mw3, mb3):
    f32 = jnp.float32
    pad = _E_PAD - _E
    src = jnp.concatenate([edge_index[0], jnp.zeros((pad,), jnp.int32)])
    src = src.reshape(_NW, _NCHUNK, _CHUNK)
    dst = jnp.concatenate([edge_index[1], jnp.full((pad,), _N, jnp.int32)])
    dst = dst.reshape(_NW, _NCHUNK, _CHUNK)
    ea = jnp.concatenate([edge_attr, jnp.zeros((pad, _EDGE_IN), f32)])
    x_pad = jnp.concatenate([x, jnp.zeros((_N_PAD - _N, _NODE_IN), f32)])

    b2r1 = e1_b2.reshape(_NODE_IN, _HID)
    b2r2 = e2_b2.reshape(_HID, _HID)

    zeros1 = jnp.zeros((_RPT, _W1C), f32)
    zeros2 = jnp.zeros((_RPT, _HID), f32)

    xs = _make_gather(_NODE_IN)(x, src)
    msg1 = _edge_messages(ea, xs, e1_w1, e1_b1.reshape(1, _HID), e1_w2, b2r1,
                          _NODE_IN, True)
    s1 = _make_scatter(_W1C)(msg1, dst, zeros1)
    h1, inv = pl.pallas_call(
        _node_update1_body,
        in_specs=[
            pl.BlockSpec((2, _N_PAD, _W1C), lambda: (0, 0, 0)),
            pl.BlockSpec((_N_PAD, _NODE_IN), lambda: (0, 0)),
            pl.BlockSpec((_NODE_IN, _HID), lambda: (0, 0)),
            pl.BlockSpec((1, _HID), lambda: (0, 0)),
        ],
        out_specs=[
            pl.BlockSpec((_N_PAD, _HID), lambda: (0, 0)),
            pl.BlockSpec((_N_PAD, 8), lambda: (0, 0)),
        ],
        out_shape=[
            jax.ShapeDtypeStruct((_N_PAD, _HID), f32),
            jax.ShapeDtypeStruct((_N_PAD, 8), f32),
        ],
    )(s1, x_pad, root1, bias1.reshape(1, _HID))

    h1s = _make_gather(_HID)(h1, src)
    msg2 = _edge_messages(ea, h1s, e2_w1, e2_b1.reshape(1, _HID), e2_w2, b2r2,
                          _HID, False)
    s2 = _make_scatter(_HID)(msg2, dst, zeros2)

    full = lambda *shape: pl.BlockSpec(shape, lambda: tuple(0 for _ in shape))
    out = pl.pallas_call(
        _head_body,
        in_specs=[
            full(2, _N_PAD, _HID),
            full(_N_PAD, _HID),
            full(_N_PAD, 8),
            full(_HID, _HID),
            full(1, _HID),
            full(_HID, _G_DIM),
            full(1, _G_DIM),
            full(_B, _N_ACT),
            full(_N_ACT + _G_DIM, _MLP_HID),
            full(1, _MLP_HID),
            full(_MLP_HID, _MLP_HID),
            full(1, _MLP_HID),
            full(_MLP_HID, 1),
            full(1, 1),
        ],
        out_specs=full(_B, 1),
        out_shape=jax.ShapeDtypeStruct((_B, 1), f32),
    )(s2, h1, inv, root2, bias2.reshape(1, _HID), proj_w,
      proj_b.reshape(1, _G_DIM), a, mw1, mb1.reshape(1, _MLP_HID), mw2,
      mb2.reshape(1, _MLP_HID), mw3, mb3.reshape(1, 1))
    return out


# speedup vs baseline: 3.1657x; 1.0279x over previous
import functools

import jax
import jax.numpy as jnp
from jax import lax
from jax.experimental import pallas as pl
from jax.experimental.pallas import tpu as pltpu
from jax.experimental.pallas import tpu_sc as plsc

_N = 10000
_E = 50000
_NODE_IN = 32
_EDGE_IN = 16
_HID = 64
_G_DIM = 64
_N_ACT = 16
_B = 1024
_MLP_HID = 128

_NW = 32
_CHUNK = 112
_NCHUNK = 14
_EPW = _CHUNK * _NCHUNK
_E_PAD = _NW * _EPW
_N_PAD = 10240
_RPT = _N_PAD // 16
_ET = 512
_W1C = _HID + 16


def _sc_mesh():
    return plsc.VectorSubcoreMesh(core_axis_name="c", subcore_axis_name="s")


def _make_gather(d):
    @functools.partial(
        pl.kernel,
        out_type=jax.ShapeDtypeStruct((_E_PAD, d), jnp.float32),
        mesh=_sc_mesh(),
        scratch_types=[
            pltpu.VMEM((_NCHUNK, _CHUNK), jnp.int32),
            pltpu.VMEM((_EPW, d), jnp.float32),
            pltpu.SemaphoreType.DMA,
        ],
        compiler_params=pltpu.CompilerParams(use_tc_tiling_on_sc=False),
    )
    def gather_k(tbl_hbm, idx_hbm, out_hbm, idx_v, rows_v, sem):
        wid = lax.axis_index("s") * 2 + lax.axis_index("c")
        pltpu.sync_copy(idx_hbm.at[wid], idx_v)
        descs = [
            pltpu.async_copy(
                tbl_hbm.at[idx_v.at[j]],
                rows_v.at[pl.ds(j * _CHUNK, _CHUNK)],
                sem,
            )
            for j in range(_NCHUNK)
        ]
        for d in descs:
            d.wait()
        pltpu.sync_copy(rows_v, out_hbm.at[pl.ds(wid * _EPW, _EPW)])

    return gather_k


def _make_scatter(w):
    @functools.partial(
        pl.kernel,
        out_type=jax.ShapeDtypeStruct((2, _N_PAD, w), jnp.float32),
        mesh=_sc_mesh(),
        scratch_types=[
            pltpu.VMEM((_NCHUNK, _CHUNK), jnp.int32),
            pltpu.VMEM((2, _CHUNK, w), jnp.float32),
            pltpu.VMEM_SHARED((_N_PAD, w), jnp.float32),
            pltpu.SemaphoreType.DMA((2,)),
        ],
        compiler_params=pltpu.CompilerParams(use_tc_tiling_on_sc=False),
    )
    def scatter_k(msg_hbm, dst_hbm, zeros_hbm, out_hbm, idx_v, buf_v, acc_sh, sem):
        cid = lax.axis_index("c")
        sid = lax.axis_index("s")
        wid = sid * 2 + cid
        pltpu.sync_copy(zeros_hbm, acc_sh.at[pl.ds(sid * _RPT, _RPT)])
        plsc.subcore_barrier()
        pltpu.sync_copy(dst_hbm.at[wid], idx_v)
        descs = [None] * _NCHUNK
        descs[0] = pltpu.async_copy(
            msg_hbm.at[pl.ds(wid * _EPW, _CHUNK)], buf_v.at[0], sem.at[0])
        for j in range(_NCHUNK):
            if j + 1 < _NCHUNK:
                descs[j + 1] = pltpu.async_copy(
                    msg_hbm.at[pl.ds(wid * _EPW + (j + 1) * _CHUNK, _CHUNK)],
                    buf_v.at[(j + 1) % 2], sem.at[(j + 1) % 2])
            descs[j].wait()
            pltpu.sync_copy(buf_v.at[j % 2], acc_sh.at[idx_v.at[j]], add=True)
        plsc.subcore_barrier()
        pltpu.sync_copy(
            acc_sh.at[pl.ds(sid * _RPT, _RPT)],
            out_hbm.at[cid, pl.ds(sid * _RPT, _RPT)],
        )

    return scatter_k


def _msg_body(in_ch, with_flag, ea_ref, xs_ref, w1_ref, b1_ref, w2_ref,
              b2r_ref, rmat_ref, out_ref):
    h = jnp.maximum(
        jnp.dot(ea_ref[...], w1_ref[...], preferred_element_type=jnp.float32)
        + b1_ref[...], 0.0)
    xs = xs_ref[...]
    hw = jnp.dot(h, w2_ref[...], preferred_element_type=jnp.float32)
    xs2 = jnp.dot(xs, rmat_ref[...], preferred_element_type=jnp.float32)
    u = xs2 * hw
    w = in_ch * _HID
    while w > 2 * _HID:
        w //= 2
        u = u[:, :w] + u[:, w:]
    msg = u[:, :_HID] + u[:, _HID:]
    msg = msg + jnp.dot(xs, b2r_ref[...], preferred_element_type=jnp.float32)
    out_ref[:, :_HID] = msg
    if with_flag:
        i = pl.program_id(0)
        rowid = i * _ET + lax.broadcasted_iota(jnp.int32, (_ET, 16), 0)
        out_ref[:, _HID:] = jnp.where(rowid < _E, 1.0, 0.0)


def _edge_messages(ea, xs, w1, b1, w2ik, b2r, in_ch, with_flag):
    outw = _W1C if with_flag else _HID
    grid = (_E_PAD // _ET,)
    return pl.pallas_call(
        functools.partial(_msg_body, in_ch, with_flag),
        grid=grid,
        in_specs=[
            pl.BlockSpec((_ET, _EDGE_IN), lambda i: (i, 0)),
            pl.BlockSpec((_ET, in_ch), lambda i: (i, 0)),
            pl.BlockSpec((_EDGE_IN, _HID), lambda i: (0, 0)),
            pl.BlockSpec((1, _HID), lambda i: (0, 0)),
            pl.BlockSpec((_HID, in_ch * _HID), lambda i: (0, 0)),
            pl.BlockSpec((in_ch, _HID), lambda i: (0, 0)),
            pl.BlockSpec((in_ch, in_ch * _HID), lambda i: (0, 0)),
        ],
        out_specs=pl.BlockSpec((_ET, outw), lambda i: (i, 0)),
        out_shape=jax.ShapeDtypeStruct((_E_PAD, outw), jnp.float32),
    )(ea, xs, w1, b1, w2ik, b2r,
      jnp.repeat(jnp.eye(in_ch, dtype=jnp.float32), _HID, axis=1))


def _node_update1_body(s_ref, x_ref, r_ref, b_ref, h_ref, inv_ref):
    s = s_ref[0] + s_ref[1]
    cnt = s[:, _HID:_HID + 1]
    inv = 1.0 / jnp.maximum(cnt, 1.0)
    mean = s[:, :_HID] * inv
    h = jnp.maximum(
        mean + jnp.dot(x_ref[...], r_ref[...], preferred_element_type=jnp.float32)
        + b_ref[...], 0.0)
    h_ref[...] = h
    inv_ref[...] = jnp.broadcast_to(inv, (_N_PAD, 8))


def _head_body(s_ref, h1_ref, inv_ref, r_ref, b_ref, pw_ref, pb_ref, a_ref,
               mw1_ref, mb1_ref, mw2_ref, mb2_ref, mw3_ref, mb3_ref, out_ref):
    s = s_ref[0] + s_ref[1]
    mean2 = s * inv_ref[:, 0:1]
    h2 = jnp.maximum(
        mean2
        + jnp.dot(h1_ref[...], r_ref[...], preferred_element_type=jnp.float32)
        + b_ref[...], 0.0)
    rid = lax.broadcasted_iota(jnp.int32, (_N_PAD, _HID), 0)
    h2 = jnp.where(rid < _N, h2, 0.0)
    pooled = jnp.sum(h2, axis=0, keepdims=True) * (1.0 / _N)
    g = jnp.dot(pooled, pw_ref[...], preferred_element_type=jnp.float32) + pb_ref[...]
    z = (jnp.dot(a_ref[...], mw1_ref[:_N_ACT, :], preferred_element_type=jnp.float32)
         + jnp.dot(g, mw1_ref[_N_ACT:, :], preferred_element_type=jnp.float32)
         + mb1_ref[...])
    z = jnp.maximum(z, 0.0)
    z = jnp.maximum(
        jnp.dot(z, mw2_ref[...], preferred_element_type=jnp.float32) + mb2_ref[...],
        0.0)
    out_ref[...] = (
        jnp.dot(z, mw3_ref[...], preferred_element_type=jnp.float32) + mb3_ref[...])


def kernel(x, edge_index, edge_attr, a,
           e1_w1, e1_b1, e1_w2, e1_b2, root1, bias1,
           e2_w1, e2_b1, e2_w2, e2_b2, root2, bias2,
           proj_w, proj_b, mw1, mb1, mw2, mb2, mw3, mb3):
    f32 = jnp.float32
    pad = _E_PAD - _E
    src = jnp.concatenate([edge_index[0], jnp.zeros((pad,), jnp.int32)])
    src = src.reshape(_NW, _NCHUNK, _CHUNK)
    dst = jnp.concatenate([edge_index[1], jnp.full((pad,), _N, jnp.int32)])
    dst = dst.reshape(_NW, _NCHUNK, _CHUNK)
    ea = jnp.concatenate([edge_attr, jnp.zeros((pad, _EDGE_IN), f32)])
    x_pad = jnp.concatenate([x, jnp.zeros((_N_PAD - _N, _NODE_IN), f32)])

    b2r1 = e1_b2.reshape(_NODE_IN, _HID)
    b2r2 = e2_b2.reshape(_HID, _HID)

    zeros1 = jnp.zeros((_RPT, _W1C), f32)
    zeros2 = jnp.zeros((_RPT, _HID), f32)

    xs = _make_gather(_NODE_IN)(x, src)
    msg1 = _edge_messages(ea, xs, e1_w1, e1_b1.reshape(1, _HID), e1_w2, b2r1,
                          _NODE_IN, True)
    s1 = _make_scatter(_W1C)(msg1, dst, zeros1)
    h1, inv = pl.pallas_call(
        _node_update1_body,
        in_specs=[
            pl.BlockSpec((2, _N_PAD, _W1C), lambda: (0, 0, 0)),
            pl.BlockSpec((_N_PAD, _NODE_IN), lambda: (0, 0)),
            pl.BlockSpec((_NODE_IN, _HID), lambda: (0, 0)),
            pl.BlockSpec((1, _HID), lambda: (0, 0)),
        ],
        out_specs=[
            pl.BlockSpec((_N_PAD, _HID), lambda: (0, 0)),
            pl.BlockSpec((_N_PAD, 8), lambda: (0, 0)),
        ],
        out_shape=[
            jax.ShapeDtypeStruct((_N_PAD, _HID), f32),
            jax.ShapeDtypeStruct((_N_PAD, 8), f32),
        ],
    )(s1, x_pad, root1, bias1.reshape(1, _HID))

    h1s = _make_gather(_HID)(h1, src)
    msg2 = _edge_messages(ea, h1s, e2_w1, e2_b1.reshape(1, _HID), e2_w2, b2r2,
                          _HID, False)
    s2 = _make_scatter(_HID)(msg2, dst, zeros2)

    full = lambda *shape: pl.BlockSpec(shape, lambda: tuple(0 for _ in shape))
    out = pl.pallas_call(
        _head_body,
        in_specs=[
            full(2, _N_PAD, _HID),
            full(_N_PAD, _HID),
            full(_N_PAD, 8),
            full(_HID, _HID),
            full(1, _HID),
            full(_HID, _G_DIM),
            full(1, _G_DIM),
            full(_B, _N_ACT),
            full(_N_ACT + _G_DIM, _MLP_HID),
            full(1, _MLP_HID),
            full(_MLP_HID, _MLP_HID),
            full(1, _MLP_HID),
            full(_MLP_HID, 1),
            full(1, 1),
        ],
        out_specs=full(_B, 1),
        out_shape=jax.ShapeDtypeStruct((_B, 1), f32),
    )(s2, h1, inv, root2, bias2.reshape(1, _HID), proj_w,
      proj_b.reshape(1, _G_DIM), a, mw1, mb1.reshape(1, _MLP_HID), mw2,
      mb2.reshape(1, _MLP_HID), mw3, mb3.reshape(1, 1))
    return out


# speedup vs baseline: 4.7610x; 1.5039x over previous
import functools

import jax
import jax.numpy as jnp
from jax import lax
from jax.experimental import pallas as pl
from jax.experimental.pallas import tpu as pltpu
from jax.experimental.pallas import tpu_sc as plsc

_N = 10000
_E = 50000
_NODE_IN = 32
_EDGE_IN = 16
_HID = 64
_G_DIM = 64
_N_ACT = 16
_B = 1024
_MLP_HID = 128

_NW = 32
_CHUNK = 112
_NCHUNK = 14
_EPW = _CHUNK * _NCHUNK
_E_PAD = _NW * _EPW
_N_PAD = 10240
_RPT = _N_PAD // 16
_ET = 512
_W1C = _HID + 16


def _sc_mesh():
    return plsc.VectorSubcoreMesh(core_axis_name="c", subcore_axis_name="s")


def _make_gather(d):
    @functools.partial(
        pl.kernel,
        out_type=jax.ShapeDtypeStruct((_E_PAD, d), jnp.float32),
        mesh=_sc_mesh(),
        scratch_types=[
            pltpu.VMEM((_NCHUNK, _CHUNK), jnp.int32),
            pltpu.VMEM((_EPW, d), jnp.float32),
            pltpu.SemaphoreType.DMA,
        ],
        compiler_params=pltpu.CompilerParams(use_tc_tiling_on_sc=False),
    )
    def gather_k(tbl_hbm, idx_hbm, out_hbm, idx_v, rows_v, sem):
        wid = lax.axis_index("s") * 2 + lax.axis_index("c")
        pltpu.sync_copy(idx_hbm.at[wid], idx_v)
        descs = [
            pltpu.async_copy(
                tbl_hbm.at[idx_v.at[j]],
                rows_v.at[pl.ds(j * _CHUNK, _CHUNK)],
                sem,
            )
            for j in range(_NCHUNK)
        ]
        for d in descs:
            d.wait()
        pltpu.sync_copy(rows_v, out_hbm.at[pl.ds(wid * _EPW, _EPW)])

    return gather_k


def _make_scatter(w):
    @functools.partial(
        pl.kernel,
        out_type=jax.ShapeDtypeStruct((2, _N_PAD, w), jnp.float32),
        mesh=_sc_mesh(),
        scratch_types=[
            pltpu.VMEM((_NCHUNK, _CHUNK), jnp.int32),
            pltpu.VMEM((2, _CHUNK, w), jnp.float32),
            pltpu.VMEM_SHARED((_N_PAD, w), jnp.float32),
            pltpu.SemaphoreType.DMA((2,)),
        ],
        compiler_params=pltpu.CompilerParams(use_tc_tiling_on_sc=False),
    )
    def scatter_k(msg_hbm, dst_hbm, zeros_hbm, out_hbm, idx_v, buf_v, acc_sh, sem):
        cid = lax.axis_index("c")
        sid = lax.axis_index("s")
        wid = sid * 2 + cid
        pltpu.sync_copy(zeros_hbm, acc_sh.at[pl.ds(sid * _RPT, _RPT)])
        plsc.subcore_barrier()
        pltpu.sync_copy(dst_hbm.at[wid], idx_v)
        descs = [None] * _NCHUNK
        descs[0] = pltpu.async_copy(
            msg_hbm.at[pl.ds(wid * _EPW, _CHUNK)], buf_v.at[0], sem.at[0])
        for j in range(_NCHUNK):
            if j + 1 < _NCHUNK:
                descs[j + 1] = pltpu.async_copy(
                    msg_hbm.at[pl.ds(wid * _EPW + (j + 1) * _CHUNK, _CHUNK)],
                    buf_v.at[(j + 1) % 2], sem.at[(j + 1) % 2])
            descs[j].wait()
            pltpu.sync_copy(buf_v.at[j % 2], acc_sh.at[idx_v.at[j]], add=True)
        plsc.subcore_barrier()
        pltpu.sync_copy(
            acc_sh.at[pl.ds(sid * _RPT, _RPT)],
            out_hbm.at[cid, pl.ds(sid * _RPT, _RPT)],
        )

    return scatter_k


def _msg_body(in_ch, with_flag, eaT_ref, xs_ref, w1T_ref, b1T_ref, w2T_ref,
              b2rT_ref, out_ref):
    hT = jnp.maximum(
        jnp.dot(w1T_ref[...], eaT_ref[...], preferred_element_type=jnp.float32)
        + b1T_ref[...], 0.0)
    xsT = xs_ref[...].T
    uT = jnp.concatenate([hT * xsT[i:i + 1, :] for i in range(in_ch)], axis=0)
    msgT = jnp.dot(w2T_ref[...], uT, preferred_element_type=jnp.float32)
    msgT = msgT + jnp.dot(b2rT_ref[...], xsT,
                          preferred_element_type=jnp.float32)
    out_ref[:, :_HID] = msgT.T
    if with_flag:
        i = pl.program_id(0)
        rowid = i * _ET + lax.broadcasted_iota(jnp.int32, (_ET, 16), 0)
        out_ref[:, _HID:] = jnp.where(rowid < _E, 1.0, 0.0)


def _edge_messages(eaT, xs, w1, b1, w2, b2, in_ch, with_flag):
    outw = _W1C if with_flag else _HID
    grid = (_E_PAD // _ET,)
    w2T = w2.reshape(_HID, in_ch, _HID).transpose(2, 1, 0)
    w2T = w2T.reshape(_HID, in_ch * _HID)
    return pl.pallas_call(
        functools.partial(_msg_body, in_ch, with_flag),
        grid=grid,
        in_specs=[
            pl.BlockSpec((_EDGE_IN, _ET), lambda i: (0, i)),
            pl.BlockSpec((_ET, in_ch), lambda i: (i, 0)),
            pl.BlockSpec((_HID, _EDGE_IN), lambda i: (0, 0)),
            pl.BlockSpec((_HID, 1), lambda i: (0, 0)),
            pl.BlockSpec((_HID, in_ch * _HID), lambda i: (0, 0)),
            pl.BlockSpec((_HID, in_ch), lambda i: (0, 0)),
        ],
        out_specs=pl.BlockSpec((_ET, outw), lambda i: (i, 0)),
        out_shape=jax.ShapeDtypeStruct((_E_PAD, outw), jnp.float32),
    )(eaT, xs, w1.T, b1.reshape(_HID, 1), w2T, b2.reshape(in_ch, _HID).T)


def _node_update1_body(s_ref, x_ref, r_ref, b_ref, h_ref, inv_ref):
    s = s_ref[0] + s_ref[1]
    cnt = s[:, _HID:_HID + 1]
    inv = 1.0 / jnp.maximum(cnt, 1.0)
    mean = s[:, :_HID] * inv
    h = jnp.maximum(
        mean + jnp.dot(x_ref[...], r_ref[...], preferred_element_type=jnp.float32)
        + b_ref[...], 0.0)
    h_ref[...] = h
    inv_ref[...] = jnp.broadcast_to(inv, (_N_PAD, 8))


def _head_body(s_ref, h1_ref, inv_ref, r_ref, b_ref, pw_ref, pb_ref, a_ref,
               mw1_ref, mb1_ref, mw2_ref, mb2_ref, mw3_ref, mb3_ref, out_ref):
    s = s_ref[0] + s_ref[1]
    mean2 = s * inv_ref[:, 0:1]
    h2 = jnp.maximum(
        mean2
        + jnp.dot(h1_ref[...], r_ref[...], preferred_element_type=jnp.float32)
        + b_ref[...], 0.0)
    rid = lax.broadcasted_iota(jnp.int32, (_N_PAD, _HID), 0)
    h2 = jnp.where(rid < _N, h2, 0.0)
    pooled = jnp.sum(h2, axis=0, keepdims=True) * (1.0 / _N)
    g = jnp.dot(pooled, pw_ref[...], preferred_element_type=jnp.float32) + pb_ref[...]
    z = (jnp.dot(a_ref[...], mw1_ref[:_N_ACT, :], preferred_element_type=jnp.float32)
         + jnp.dot(g, mw1_ref[_N_ACT:, :], preferred_element_type=jnp.float32)
         + mb1_ref[...])
    z = jnp.maximum(z, 0.0)
    z = jnp.maximum(
        jnp.dot(z, mw2_ref[...], preferred_element_type=jnp.float32) + mb2_ref[...],
        0.0)
    out_ref[...] = (
        jnp.dot(z, mw3_ref[...], preferred_element_type=jnp.float32) + mb3_ref[...])


def kernel(x, edge_index, edge_attr, a,
           e1_w1, e1_b1, e1_w2, e1_b2, root1, bias1,
           e2_w1, e2_b1, e2_w2, e2_b2, root2, bias2,
           proj_w, proj_b, mw1, mb1, mw2, mb2, mw3, mb3):
    f32 = jnp.float32
    pad = _E_PAD - _E
    src = jnp.concatenate([edge_index[0], jnp.zeros((pad,), jnp.int32)])
    src = src.reshape(_NW, _NCHUNK, _CHUNK)
    dst = jnp.concatenate([edge_index[1], jnp.full((pad,), _N, jnp.int32)])
    dst = dst.reshape(_NW, _NCHUNK, _CHUNK)
    eaT = jnp.concatenate([edge_attr, jnp.zeros((pad, _EDGE_IN), f32)]).T
    x_pad = jnp.concatenate([x, jnp.zeros((_N_PAD - _N, _NODE_IN), f32)])


    zeros1 = jnp.zeros((_RPT, _W1C), f32)
    zeros2 = jnp.zeros((_RPT, _HID), f32)

    xs = _make_gather(_NODE_IN)(x, src)
    msg1 = _edge_messages(eaT, xs, e1_w1, e1_b1, e1_w2, e1_b2,
                          _NODE_IN, True)
    s1 = _make_scatter(_W1C)(msg1, dst, zeros1)
    h1, inv = pl.pallas_call(
        _node_update1_body,
        in_specs=[
            pl.BlockSpec((2, _N_PAD, _W1C), lambda: (0, 0, 0)),
            pl.BlockSpec((_N_PAD, _NODE_IN), lambda: (0, 0)),
            pl.BlockSpec((_NODE_IN, _HID), lambda: (0, 0)),
            pl.BlockSpec((1, _HID), lambda: (0, 0)),
        ],
        out_specs=[
            pl.BlockSpec((_N_PAD, _HID), lambda: (0, 0)),
            pl.BlockSpec((_N_PAD, 8), lambda: (0, 0)),
        ],
        out_shape=[
            jax.ShapeDtypeStruct((_N_PAD, _HID), f32),
            jax.ShapeDtypeStruct((_N_PAD, 8), f32),
        ],
    )(s1, x_pad, root1, bias1.reshape(1, _HID))

    h1s = _make_gather(_HID)(h1, src)
    msg2 = _edge_messages(eaT, h1s, e2_w1, e2_b1, e2_w2, e2_b2,
                          _HID, False)
    s2 = _make_scatter(_HID)(msg2, dst, zeros2)

    full = lambda *shape: pl.BlockSpec(shape, lambda: tuple(0 for _ in shape))
    out = pl.pallas_call(
        _head_body,
        in_specs=[
            full(2, _N_PAD, _HID),
            full(_N_PAD, _HID),
            full(_N_PAD, 8),
            full(_HID, _HID),
            full(1, _HID),
            full(_HID, _G_DIM),
            full(1, _G_DIM),
            full(_B, _N_ACT),
            full(_N_ACT + _G_DIM, _MLP_HID),
            full(1, _MLP_HID),
            full(_MLP_HID, _MLP_HID),
            full(1, _MLP_HID),
            full(_MLP_HID, 1),
            full(1, 1),
        ],
        out_specs=full(_B, 1),
        out_shape=jax.ShapeDtypeStruct((_B, 1), f32),
    )(s2, h1, inv, root2, bias2.reshape(1, _HID), proj_w,
      proj_b.reshape(1, _G_DIM), a, mw1, mb1.reshape(1, _MLP_HID), mw2,
      mb2.reshape(1, _MLP_HID), mw3, mb3.reshape(1, 1))
    return out


# speedup vs baseline: 5.4660x; 1.1481x over previous
import functools

import jax
import jax.numpy as jnp
from jax import lax
from jax.experimental import pallas as pl
from jax.experimental.pallas import tpu as pltpu
from jax.experimental.pallas import tpu_sc as plsc

_N = 10000
_E = 50000
_NODE_IN = 32
_EDGE_IN = 16
_HID = 64
_G_DIM = 64
_N_ACT = 16
_B = 1024
_MLP_HID = 128

_NW = 32
_CHUNK = 112
_NCHUNK = 14
_EPW = _CHUNK * _NCHUNK
_E_PAD = _NW * _EPW
_N_PAD = 10240
_RPT = _N_PAD // 16
_ET = 1024
_W1C = _HID + 16


def _sc_mesh():
    return plsc.VectorSubcoreMesh(core_axis_name="c", subcore_axis_name="s")


def _make_gather(d):
    @functools.partial(
        pl.kernel,
        out_type=jax.ShapeDtypeStruct((_E_PAD, d), jnp.float32),
        mesh=_sc_mesh(),
        scratch_types=[
            pltpu.VMEM((_NCHUNK, _CHUNK), jnp.int32),
            pltpu.VMEM((_EPW, d), jnp.float32),
            pltpu.SemaphoreType.DMA((_NCHUNK,)),
            pltpu.SemaphoreType.DMA,
        ],
        compiler_params=pltpu.CompilerParams(use_tc_tiling_on_sc=False),
    )
    def gather_k(tbl_hbm, idx_hbm, out_hbm, idx_v, rows_v, sem, sem_out):
        wid = lax.axis_index("s") * 2 + lax.axis_index("c")
        pltpu.sync_copy(idx_hbm.at[wid], idx_v)
        descs = [
            pltpu.async_copy(
                tbl_hbm.at[idx_v.at[j]],
                rows_v.at[pl.ds(j * _CHUNK, _CHUNK)],
                sem.at[j],
            )
            for j in range(_NCHUNK)
        ]
        outs = []
        for j in range(_NCHUNK):
            descs[j].wait()
            outs.append(pltpu.async_copy(
                rows_v.at[pl.ds(j * _CHUNK, _CHUNK)],
                out_hbm.at[pl.ds(wid * _EPW + j * _CHUNK, _CHUNK)],
                sem_out,
            ))
        for d in outs:
            d.wait()

    return gather_k


def _make_scatter(w):
    @functools.partial(
        pl.kernel,
        out_type=jax.ShapeDtypeStruct((2, _N_PAD, w), jnp.float32),
        mesh=_sc_mesh(),
        scratch_types=[
            pltpu.VMEM((_NCHUNK, _CHUNK), jnp.int32),
            pltpu.VMEM((2, _CHUNK, w), jnp.float32),
            pltpu.VMEM_SHARED((_N_PAD, w), jnp.float32),
            pltpu.SemaphoreType.DMA((2,)),
        ],
        compiler_params=pltpu.CompilerParams(use_tc_tiling_on_sc=False),
    )
    def scatter_k(msg_hbm, dst_hbm, zeros_hbm, out_hbm, idx_v, buf_v, acc_sh, sem):
        cid = lax.axis_index("c")
        sid = lax.axis_index("s")
        wid = sid * 2 + cid
        pltpu.sync_copy(zeros_hbm, acc_sh.at[pl.ds(sid * _RPT, _RPT)])
        plsc.subcore_barrier()
        pltpu.sync_copy(dst_hbm.at[wid], idx_v)
        descs = [None] * _NCHUNK
        descs[0] = pltpu.async_copy(
            msg_hbm.at[pl.ds(wid * _EPW, _CHUNK)], buf_v.at[0], sem.at[0])
        for j in range(_NCHUNK):
            if j + 1 < _NCHUNK:
                descs[j + 1] = pltpu.async_copy(
                    msg_hbm.at[pl.ds(wid * _EPW + (j + 1) * _CHUNK, _CHUNK)],
                    buf_v.at[(j + 1) % 2], sem.at[(j + 1) % 2])
            descs[j].wait()
            pltpu.sync_copy(buf_v.at[j % 2], acc_sh.at[idx_v.at[j]], add=True)
        plsc.subcore_barrier()
        pltpu.sync_copy(
            acc_sh.at[pl.ds(sid * _RPT, _RPT)],
            out_hbm.at[cid, pl.ds(sid * _RPT, _RPT)],
        )

    return scatter_k


def _msg_body(in_ch, with_flag, eaT_ref, xs_ref, w1T_ref, b1T_ref, w2T_ref,
              b2rT_ref, out_ref):
    hT = jnp.maximum(
        jnp.dot(w1T_ref[...], eaT_ref[...], preferred_element_type=jnp.float32)
        + b1T_ref[...], 0.0)
    xsT = xs_ref[...].T
    uT = jnp.concatenate([hT * xsT[i:i + 1, :] for i in range(in_ch)], axis=0)
    msgT = jnp.dot(w2T_ref[...], uT, preferred_element_type=jnp.float32)
    msgT = msgT + jnp.dot(b2rT_ref[...], xsT,
                          preferred_element_type=jnp.float32)
    out_ref[:, :_HID] = msgT.T
    if with_flag:
        i = pl.program_id(0)
        rowid = i * _ET + lax.broadcasted_iota(jnp.int32, (_ET, 16), 0)
        out_ref[:, _HID:] = jnp.where(rowid < _E, 1.0, 0.0)


def _edge_messages(eaT, xs, w1, b1, w2, b2, in_ch, with_flag):
    outw = _W1C if with_flag else _HID
    grid = (_E_PAD // _ET,)
    w2T = w2.reshape(_HID, in_ch, _HID).transpose(2, 1, 0)
    w2T = w2T.reshape(_HID, in_ch * _HID)
    return pl.pallas_call(
        functools.partial(_msg_body, in_ch, with_flag),
        grid=grid,
        in_specs=[
            pl.BlockSpec((_EDGE_IN, _ET), lambda i: (0, i)),
            pl.BlockSpec((_ET, in_ch), lambda i: (i, 0)),
            pl.BlockSpec((_HID, _EDGE_IN), lambda i: (0, 0)),
            pl.BlockSpec((_HID, 1), lambda i: (0, 0)),
            pl.BlockSpec((_HID, in_ch * _HID), lambda i: (0, 0)),
            pl.BlockSpec((_HID, in_ch), lambda i: (0, 0)),
        ],
        out_specs=pl.BlockSpec((_ET, outw), lambda i: (i, 0)),
        out_shape=jax.ShapeDtypeStruct((_E_PAD, outw), jnp.float32),
    )(eaT, xs, w1.T, b1.reshape(_HID, 1), w2T, b2.reshape(in_ch, _HID).T)


def _node_update1_body(s_ref, x_ref, r_ref, b_ref, h_ref, inv_ref):
    s = s_ref[0] + s_ref[1]
    cnt = s[:, _HID:_HID + 1]
    inv = 1.0 / jnp.maximum(cnt, 1.0)
    mean = s[:, :_HID] * inv
    h = jnp.maximum(
        mean + jnp.dot(x_ref[...], r_ref[...], preferred_element_type=jnp.float32)
        + b_ref[...], 0.0)
    h_ref[...] = h
    inv_ref[...] = jnp.broadcast_to(inv, (_N_PAD, 8))


def _head_body(s_ref, h1_ref, inv_ref, r_ref, b_ref, pw_ref, pb_ref, a_ref,
               mw1_ref, mb1_ref, mw2_ref, mb2_ref, mw3_ref, mb3_ref, out_ref):
    s = s_ref[0] + s_ref[1]
    mean2 = s * inv_ref[:, 0:1]
    h2 = jnp.maximum(
        mean2
        + jnp.dot(h1_ref[...], r_ref[...], preferred_element_type=jnp.float32)
        + b_ref[...], 0.0)
    rid = lax.broadcasted_iota(jnp.int32, (_N_PAD, _HID), 0)
    h2 = jnp.where(rid < _N, h2, 0.0)
    pooled = jnp.sum(h2, axis=0, keepdims=True) * (1.0 / _N)
    g = jnp.dot(pooled, pw_ref[...], preferred_element_type=jnp.float32) + pb_ref[...]
    z = (jnp.dot(a_ref[...], mw1_ref[:_N_ACT, :], preferred_element_type=jnp.float32)
         + jnp.dot(g, mw1_ref[_N_ACT:, :], preferred_element_type=jnp.float32)
         + mb1_ref[...])
    z = jnp.maximum(z, 0.0)
    z = jnp.maximum(
        jnp.dot(z, mw2_ref[...], preferred_element_type=jnp.float32) + mb2_ref[...],
        0.0)
    out_ref[...] = (
        jnp.dot(z, mw3_ref[...], preferred_element_type=jnp.float32) + mb3_ref[...])


def kernel(x, edge_index, edge_attr, a,
           e1_w1, e1_b1, e1_w2, e1_b2, root1, bias1,
           e2_w1, e2_b1, e2_w2, e2_b2, root2, bias2,
           proj_w, proj_b, mw1, mb1, mw2, mb2, mw3, mb3):
    f32 = jnp.float32
    pad = _E_PAD - _E
    src = jnp.concatenate([edge_index[0], jnp.zeros((pad,), jnp.int32)])
    src = src.reshape(_NW, _NCHUNK, _CHUNK)
    dst = jnp.concatenate([edge_index[1], jnp.full((pad,), _N, jnp.int32)])
    dst = dst.reshape(_NW, _NCHUNK, _CHUNK)
    eaT = jnp.concatenate([edge_attr, jnp.zeros((pad, _EDGE_IN), f32)]).T
    x_pad = jnp.concatenate([x, jnp.zeros((_N_PAD - _N, _NODE_IN), f32)])


    zeros1 = jnp.zeros((_RPT, _W1C), f32)
    zeros2 = jnp.zeros((_RPT, _HID), f32)

    xs = _make_gather(_NODE_IN)(x, src)
    msg1 = _edge_messages(eaT, xs, e1_w1, e1_b1, e1_w2, e1_b2,
                          _NODE_IN, True)
    s1 = _make_scatter(_W1C)(msg1, dst, zeros1)
    h1, inv = pl.pallas_call(
        _node_update1_body,
        in_specs=[
            pl.BlockSpec((2, _N_PAD, _W1C), lambda: (0, 0, 0)),
            pl.BlockSpec((_N_PAD, _NODE_IN), lambda: (0, 0)),
            pl.BlockSpec((_NODE_IN, _HID), lambda: (0, 0)),
            pl.BlockSpec((1, _HID), lambda: (0, 0)),
        ],
        out_specs=[
            pl.BlockSpec((_N_PAD, _HID), lambda: (0, 0)),
            pl.BlockSpec((_N_PAD, 8), lambda: (0, 0)),
        ],
        out_shape=[
            jax.ShapeDtypeStruct((_N_PAD, _HID), f32),
            jax.ShapeDtypeStruct((_N_PAD, 8), f32),
        ],
    )(s1, x_pad, root1, bias1.reshape(1, _HID))

    h1s = _make_gather(_HID)(h1, src)
    msg2 = _edge_messages(eaT, h1s, e2_w1, e2_b1, e2_w2, e2_b2,
                          _HID, False)
    s2 = _make_scatter(_HID)(msg2, dst, zeros2)

    full = lambda *shape: pl.BlockSpec(shape, lambda: tuple(0 for _ in shape))
    out = pl.pallas_call(
        _head_body,
        in_specs=[
            full(2, _N_PAD, _HID),
            full(_N_PAD, _HID),
            full(_N_PAD, 8),
            full(_HID, _HID),
            full(1, _HID),
            full(_HID, _G_DIM),
            full(1, _G_DIM),
            full(_B, _N_ACT),
            full(_N_ACT + _G_DIM, _MLP_HID),
            full(1, _MLP_HID),
            full(_MLP_HID, _MLP_HID),
            full(1, _MLP_HID),
            full(_MLP_HID, 1),
            full(1, 1),
        ],
        out_specs=full(_B, 1),
        out_shape=jax.ShapeDtypeStruct((_B, 1), f32),
    )(s2, h1, inv, root2, bias2.reshape(1, _HID), proj_w,
      proj_b.reshape(1, _G_DIM), a, mw1, mb1.reshape(1, _MLP_HID), mw2,
      mb2.reshape(1, _MLP_HID), mw3, mb3.reshape(1, 1))
    return out
